# baseline (speedup 1.0000x reference)
import functools

import jax
import jax.numpy as jnp
from jax import lax
from jax.experimental import pallas as pl
from jax.experimental.pallas import tpu as pltpu
from jax.experimental.pallas import tpu_sc as plsc

_B = 32
_T = 64
_IN_CH = 263
_CH = 256
_ZCH = 512
_NB = 768

_BF = jnp.bfloat16


def _mm(a, b):
    return lax.dot_general(a.astype(_BF), b.astype(_BF),
                           (((1,), (0,)), ((), ())),
                           preferred_element_type=jnp.float32)


def _mm3(h, w):
    bb, tt, ci = h.shape
    return _mm(h.reshape(bb * tt, ci), w).reshape(bb, tt, w.shape[1])


def _shift_fwd(h):
    return jnp.concatenate([jnp.zeros_like(h[:, :1]), h[:, :-1]], axis=1)


def _shift_bwd(h):
    return jnp.concatenate([h[:, 1:], jnp.zeros_like(h[:, :1])], axis=1)


def _nb(ps, m):
    return ps[m - 1] if m >= 1 else _shift_fwd(ps[-1])


def _na(ps, m):
    return ps[m + 1] if m + 1 < len(ps) else _shift_bwd(ps[0])


def _conv3_p(ps, wt, b):
    bb = b[None, None, :]
    return [_mm3(_nb(ps, m), wt[0]) + _mm3(ps[m], wt[1])
            + _mm3(_na(ps, m), wt[2]) + bb
            for m in range(len(ps))]


def _relu_p(ps):
    return [jnp.maximum(h, 0.0) for h in ps]


def _resblock_p(ps, w1, b1, w2, b2, ws=None, bs=None):
    us = _conv3_p(_relu_p(ps), w1, b1)
    us = _conv3_p(_relu_p(us), w2, b2)
    if ws is not None:
        ps = [_mm3(h, ws) + bs[None, None, :] for h in ps]
    return [h + u for h, u in zip(ps, us)]


def _downconv_p(ps, wt, b):
    bb = b[None, None, :]
    return [_mm3(ps[2 * m], wt[0]) + _mm3(ps[2 * m + 1], wt[1])
            + _mm3(_na(ps, 2 * m + 1), wt[2]) + bb
            for m in range(len(ps) // 2)]


def _upconv_p(ps, wt, b):
    bb = b[None, None, :]
    out = []
    for q in range(len(ps)):
        out.append(_mm3(_nb(ps, q), wt[0]) + _mm3(ps[q], wt[1])
                   + _mm3(ps[q], wt[2]) + bb)
        out.append(_mm3(ps[q], wt[0]) + _mm3(ps[q], wt[1])
                   + _mm3(_na(ps, q), wt[2]) + bb)
    return out


def _call(body, args, out_shapes):
    return pl.pallas_call(
        body,
        out_shape=[jax.ShapeDtypeStruct(s, d) for (s, d) in out_shapes],
    )(*args)


def _enc_body(x0, x1, x2, x3, w_in, b_in,
              w001, b001, w002, b002, w011, b011, w012, b012,
              wd0, bd0,
              w101, b101, w102, b102, ws10, bs10, w111, b111, w112, b112,
              wd1, bd1,
              w201, b201, w202, b202, ws20, bs20, w211, b211, w212, b212,
              wo, bo, cbt, idx_out):
    ps = [x0[...], x1[...], x2[...], x3[...]]
    ps = _conv3_p(ps, w_in[...], b_in[...])
    ps = _resblock_p(ps, w001[...], b001[...], w002[...], b002[...])
    ps = _resblock_p(ps, w011[...], b011[...], w012[...], b012[...])
    ps = _downconv_p(ps, wd0[...], bd0[...])
    ps = _resblock_p(ps, w101[...], b101[...], w102[...], b102[...],
                     ws10[...], bs10[...])
    ps = _resblock_p(ps, w111[...], b111[...], w112[...], b112[...])
    ps = _downconv_p(ps, wd1[...], bd1[...])
    ps = _resblock_p(ps, w201[...], b201[...], w202[...], b202[...],
                     ws20[...], bs20[...])
    ps = _resblock_p(ps, w211[...], b211[...], w212[...], b212[...])
    z = _conv3_p(_relu_p(ps), wo[...], bo[...])[0]
    bb, tt, ci = z.shape
    zf = z.reshape(bb * tt, ci)
    cbt_v = cbt[...]
    s = _mm(zf, cbt_v)
    cn = jnp.sum(cbt_v * cbt_v, axis=0)
    d = cn[None, :] - 2.0 * s
    m = jnp.min(d, axis=1, keepdims=True)
    cols = lax.broadcasted_iota(jnp.int32, d.shape, 1)
    idx = jnp.min(jnp.where(d == m, cols, _NB), axis=1)
    idx_out[...] = idx.astype(jnp.int32)


def _dec2_body(zq, w_in, b_in,
               w201, b201, w202, b202, w211, b211, w212, b212,
               wu2, bu2, he_out, ho_out):
    ps = [zq[...]]
    ps = _conv3_p(ps, w_in[...], b_in[...])
    ps = _resblock_p(ps, w201[...], b201[...], w202[...], b202[...])
    ps = _resblock_p(ps, w211[...], b211[...], w212[...], b212[...])
    ps = _upconv_p(ps, wu2[...], bu2[...])
    he_out[...] = ps[0]
    ho_out[...] = ps[1]


def _dec01_body(he, ho,
                w101, b101, w102, b102, ws10, bs10, w111, b111, w112, b112,
                wu1, bu1,
                w001, b001, w002, b002, ws00, bs00, w011, b011, w012, b012,
                wo, bo, y0, y1, y2, y3):
    ps = [he[...], ho[...]]
    ps = _resblock_p(ps, w101[...], b101[...], w102[...], b102[...],
                     ws10[...], bs10[...])
    ps = _resblock_p(ps, w111[...], b111[...], w112[...], b112[...])
    ps = _upconv_p(ps, wu1[...], bu1[...])
    ps = _resblock_p(ps, w001[...], b001[...], w002[...], b002[...],
                     ws00[...], bs00[...])
    ps = _resblock_p(ps, w011[...], b011[...], w012[...], b012[...])
    ps = _conv3_p(_relu_p(ps), wo[...], bo[...])
    y0[...] = ps[0]
    y1[...] = ps[1]
    y2[...] = ps[2]
    y3[...] = ps[3]


def _vq_gather(cb, idx):
    info = plsc.get_sparse_core_info()
    nc, ns = info.num_cores, info.num_subcores
    nw = nc * ns
    rows = idx.shape[0]
    b_per_w = rows // nw
    mesh = plsc.VectorSubcoreMesh(core_axis_name="c", subcore_axis_name="s")

    @functools.partial(
        pl.kernel, mesh=mesh,
        out_type=jax.ShapeDtypeStruct((rows, cb.shape[1]), jnp.float32),
        scratch_types=[
            pltpu.VMEM((b_per_w,), jnp.int32),
            pltpu.VMEM((b_per_w, cb.shape[1]), jnp.float32),
            pltpu.SemaphoreType.DMA,
        ],
    )
    def gather_k(table_hbm, idx_hbm, out_hbm, idx_v, rows_v, sem):
        wid = lax.axis_index("s") * nc + lax.axis_index("c")
        base = wid * b_per_w
        pltpu.sync_copy(idx_hbm.at[pl.ds(base, b_per_w)], idx_v)
        pltpu.async_copy(table_hbm.at[idx_v], rows_v, sem).wait()
        pltpu.sync_copy(rows_v, out_hbm.at[pl.ds(base, b_per_w)])

    return gather_k(cb, idx)


def _wt3(w):
    co, ci, _ = w.shape
    wt2 = w.reshape(co, 3 * ci).T.astype(_BF)
    wt2 = jax.lax.optimization_barrier(wt2)
    return wt2.reshape(ci, 3, co).transpose(1, 0, 2)


def _wt(p, name):
    return _wt3(p[name + '_w'])


def _res_args(p, pre, shortcut):
    a = [_wt3(p[pre + '_w1']), p[pre + '_b1'],
         _wt3(p[pre + '_w2']), p[pre + '_b2']]
    if shortcut:
        a += [p[pre + '_ws'][:, :, 0].T.astype(_BF), p[pre + '_bs']]
    return a


def kernel(x, params):
    p = params
    f32 = jnp.float32
    tq = _T // 4

    idx, = _call(
        _enc_body,
        [x[:, 0::4], x[:, 1::4], x[:, 2::4], x[:, 3::4],
         _wt(p, 'enc_in'), p['enc_in_b']]
        + _res_args(p, 'enc_r0_0', False)
        + _res_args(p, 'enc_r0_1', False)
        + [_wt(p, 'enc_d0'), p['enc_d0_b']]
        + _res_args(p, 'enc_r1_0', True)
        + _res_args(p, 'enc_r1_1', False)
        + [_wt(p, 'enc_d1'), p['enc_d1_b']]
        + _res_args(p, 'enc_r2_0', True)
        + _res_args(p, 'enc_r2_1', False)
        + [_wt(p, 'enc_out'), p['enc_out_b'], p['codebook'].T],
        [((_B * _T // 4,), jnp.int32)])

    zq = _vq_gather(params['codebook'], idx)

    he, ho = _call(
        _dec2_body,
        [zq.reshape(_B, _T // 4, _ZCH), _wt(p, 'dec_in'), p['dec_in_b']]
        + _res_args(p, 'dec_r2_0', False)
        + _res_args(p, 'dec_r2_1', False)
        + [_wt(p, 'dec_u2'), p['dec_u2_b']],
        [((_B, tq, 4 * _CH), f32)] * 2)

    ys = _call(
        _dec01_body,
        [he, ho]
        + _res_args(p, 'dec_r1_0', True)
        + _res_args(p, 'dec_r1_1', False)
        + [_wt(p, 'dec_u1'), p['dec_u1_b']]
        + _res_args(p, 'dec_r0_0', True)
        + _res_args(p, 'dec_r0_1', False)
        + [_wt(p, 'dec_out'), p['dec_out_b']],
        [((_B, tq, _IN_CH), f32)] * 4)

    y = jnp.stack(ys, axis=2).reshape(_B, _T, _IN_CH)
    return jnp.transpose(y, (0, 2, 1))


# speedup vs baseline: 2.7405x; 2.7405x over previous
import functools

import jax
import jax.numpy as jnp
from jax import lax
from jax.experimental import pallas as pl
from jax.experimental.pallas import tpu as pltpu
from jax.experimental.pallas import tpu_sc as plsc

_B = 32
_T = 64
_IN_CH = 263
_CH = 256
_ZCH = 512
_NB = 768

_BF = jnp.bfloat16


def _mm(a, b):
    return lax.dot_general(a.astype(_BF), b.astype(_BF),
                           (((1,), (0,)), ((), ())),
                           preferred_element_type=jnp.float32)


def _mm3(h, w):
    bb, tt, ci = h.shape
    return _mm(h.reshape(bb * tt, ci), w).reshape(bb, tt, w.shape[1])


def _shift_fwd(h):
    return jnp.concatenate([jnp.zeros_like(h[:, :1]), h[:, :-1]], axis=1)


def _shift_bwd(h):
    return jnp.concatenate([h[:, 1:], jnp.zeros_like(h[:, :1])], axis=1)


def _nb(ps, m):
    return ps[m - 1] if m >= 1 else _shift_fwd(ps[-1])


def _na(ps, m):
    return ps[m + 1] if m + 1 < len(ps) else _shift_bwd(ps[0])


def _conv3_p(ps, wt, b):
    bb = b[None, None, :]
    return [_mm3(_nb(ps, m), wt[0]) + _mm3(ps[m], wt[1])
            + _mm3(_na(ps, m), wt[2]) + bb
            for m in range(len(ps))]


def _relu_p(ps):
    return [jnp.maximum(h, 0.0) for h in ps]


def _resblock_p(ps, w1, b1, w2, b2, ws=None, bs=None):
    us = _conv3_p(_relu_p(ps), w1, b1)
    us = _conv3_p(_relu_p(us), w2, b2)
    if ws is not None:
        ps = [_mm3(h, ws) + bs[None, None, :] for h in ps]
    return [h + u for h, u in zip(ps, us)]


def _downconv_p(ps, wt, b):
    bb = b[None, None, :]
    return [_mm3(ps[2 * m], wt[0]) + _mm3(ps[2 * m + 1], wt[1])
            + _mm3(_na(ps, 2 * m + 1), wt[2]) + bb
            for m in range(len(ps) // 2)]


def _upconv_p(ps, wt, b):
    bb = b[None, None, :]
    out = []
    for q in range(len(ps)):
        out.append(_mm3(_nb(ps, q), wt[0]) + _mm3(ps[q], wt[1])
                   + _mm3(ps[q], wt[2]) + bb)
        out.append(_mm3(ps[q], wt[0]) + _mm3(ps[q], wt[1])
                   + _mm3(_na(ps, q), wt[2]) + bb)
    return out


def _call(body, args, out_shapes):
    return pl.pallas_call(
        body,
        out_shape=[jax.ShapeDtypeStruct(s, d) for (s, d) in out_shapes],
    )(*args)


def _enc_body(x0, x1, x2, x3, w_in, b_in,
              w001, b001, w002, b002, w011, b011, w012, b012,
              wd0, bd0,
              w101, b101, w102, b102, ws10, bs10, w111, b111, w112, b112,
              wd1, bd1,
              w201, b201, w202, b202, ws20, bs20, w211, b211, w212, b212,
              wo, bo, cbt, idx_out):
    ps = [x0[...], x1[...], x2[...], x3[...]]
    ps = _conv3_p(ps, w_in[...], b_in[...])
    ps = _resblock_p(ps, w001[...], b001[...], w002[...], b002[...])
    ps = _resblock_p(ps, w011[...], b011[...], w012[...], b012[...])
    ps = _downconv_p(ps, wd0[...], bd0[...])
    ps = _resblock_p(ps, w101[...], b101[...], w102[...], b102[...],
                     ws10[...], bs10[...])
    ps = _resblock_p(ps, w111[...], b111[...], w112[...], b112[...])
    ps = _downconv_p(ps, wd1[...], bd1[...])
    ps = _resblock_p(ps, w201[...], b201[...], w202[...], b202[...],
                     ws20[...], bs20[...])
    ps = _resblock_p(ps, w211[...], b211[...], w212[...], b212[...])
    z = _conv3_p(_relu_p(ps), wo[...], bo[...])[0]
    bb, tt, ci = z.shape
    zf = z.reshape(bb * tt, ci)
    cbt_v = cbt[...]
    s = _mm(zf, cbt_v)
    cn = jnp.sum(cbt_v * cbt_v, axis=0)
    d = cn[None, :] - 2.0 * s
    m = jnp.min(d, axis=1, keepdims=True)
    cols = lax.broadcasted_iota(jnp.int32, d.shape, 1)
    idx = jnp.min(jnp.where(d == m, cols, _NB), axis=1)
    idx_out[...] = idx.astype(jnp.int32)


def _dec2_body(zq, w_in, b_in,
               w201, b201, w202, b202, w211, b211, w212, b212,
               wu2, bu2, he_out, ho_out):
    ps = [zq[...]]
    ps = _conv3_p(ps, w_in[...], b_in[...])
    ps = _resblock_p(ps, w201[...], b201[...], w202[...], b202[...])
    ps = _resblock_p(ps, w211[...], b211[...], w212[...], b212[...])
    ps = _upconv_p(ps, wu2[...], bu2[...])
    he_out[...] = ps[0]
    ho_out[...] = ps[1]


def _dec01_body(he, ho,
                w101, b101, w102, b102, ws10, bs10, w111, b111, w112, b112,
                wu1, bu1,
                w001, b001, w002, b002, ws00, bs00, w011, b011, w012, b012,
                wo, bo, y0, y1, y2, y3):
    ps = [he[...], ho[...]]
    ps = _resblock_p(ps, w101[...], b101[...], w102[...], b102[...],
                     ws10[...], bs10[...])
    ps = _resblock_p(ps, w111[...], b111[...], w112[...], b112[...])
    ps = _upconv_p(ps, wu1[...], bu1[...])
    ps = _resblock_p(ps, w001[...], b001[...], w002[...], b002[...],
                     ws00[...], bs00[...])
    ps = _resblock_p(ps, w011[...], b011[...], w012[...], b012[...])
    ps = _conv3_p(_relu_p(ps), wo[...], bo[...])
    y0[...] = ps[0]
    y1[...] = ps[1]
    y2[...] = ps[2]
    y3[...] = ps[3]


def _vq_gather(cb, idx):
    info = plsc.get_sparse_core_info()
    nc, ns = info.num_cores, info.num_subcores
    nw = nc * ns
    rows = idx.shape[0]
    b_per_w = rows // nw
    mesh = plsc.VectorSubcoreMesh(core_axis_name="c", subcore_axis_name="s")

    @functools.partial(
        pl.kernel, mesh=mesh,
        out_type=jax.ShapeDtypeStruct((rows, cb.shape[1]), jnp.float32),
        scratch_types=[
            pltpu.VMEM((b_per_w,), jnp.int32),
            pltpu.VMEM((b_per_w, cb.shape[1]), jnp.float32),
            pltpu.SemaphoreType.DMA,
        ],
    )
    def gather_k(table_hbm, idx_hbm, out_hbm, idx_v, rows_v, sem):
        wid = lax.axis_index("s") * nc + lax.axis_index("c")
        base = wid * b_per_w
        pltpu.sync_copy(idx_hbm.at[pl.ds(base, b_per_w)], idx_v)
        pltpu.async_copy(table_hbm.at[idx_v], rows_v, sem).wait()
        pltpu.sync_copy(rows_v, out_hbm.at[pl.ds(base, b_per_w)])

    return gather_k(cb, idx)


def _wt(p, name):
    return jnp.transpose(p[name + '_w'], (2, 1, 0)).astype(_BF)


def _res_args(p, pre, shortcut):
    a = [jnp.transpose(p[pre + '_w1'], (2, 1, 0)).astype(_BF), p[pre + '_b1'],
         jnp.transpose(p[pre + '_w2'], (2, 1, 0)).astype(_BF), p[pre + '_b2']]
    if shortcut:
        a += [p[pre + '_ws'][:, :, 0].T.astype(_BF), p[pre + '_bs']]
    return a


def kernel(x, params):
    p = params
    f32 = jnp.float32
    tq = _T // 4

    idx, = _call(
        _enc_body,
        [x[:, 0::4], x[:, 1::4], x[:, 2::4], x[:, 3::4],
         _wt(p, 'enc_in'), p['enc_in_b']]
        + _res_args(p, 'enc_r0_0', False)
        + _res_args(p, 'enc_r0_1', False)
        + [_wt(p, 'enc_d0'), p['enc_d0_b']]
        + _res_args(p, 'enc_r1_0', True)
        + _res_args(p, 'enc_r1_1', False)
        + [_wt(p, 'enc_d1'), p['enc_d1_b']]
        + _res_args(p, 'enc_r2_0', True)
        + _res_args(p, 'enc_r2_1', False)
        + [_wt(p, 'enc_out'), p['enc_out_b'], p['codebook'].T],
        [((_B * _T // 4,), jnp.int32)])

    zq = _vq_gather(params['codebook'], idx)

    he, ho = _call(
        _dec2_body,
        [zq.reshape(_B, _T // 4, _ZCH), _wt(p, 'dec_in'), p['dec_in_b']]
        + _res_args(p, 'dec_r2_0', False)
        + _res_args(p, 'dec_r2_1', False)
        + [_wt(p, 'dec_u2'), p['dec_u2_b']],
        [((_B, tq, 4 * _CH), f32)] * 2)

    ys = _call(
        _dec01_body,
        [he, ho]
        + _res_args(p, 'dec_r1_0', True)
        + _res_args(p, 'dec_r1_1', False)
        + [_wt(p, 'dec_u1'), p['dec_u1_b']]
        + _res_args(p, 'dec_r0_0', True)
        + _res_args(p, 'dec_r0_1', False)
        + [_wt(p, 'dec_out'), p['dec_out_b']],
        [((_B, tq, _IN_CH), f32)] * 4)

    y = jnp.stack(ys, axis=2).reshape(_B, _T, _IN_CH)
    return jnp.transpose(y, (0, 2, 1))
